# baseline (speedup 1.0000x reference)
import functools

import jax
import jax.numpy as jnp
from jax import lax
from jax.experimental import pallas as pl
from jax.experimental.pallas import tpu as pltpu
from jax.experimental.pallas import tpu_sc as plsc

N = 10000
D = 128
DH = D // 2
E = 320000
NC = 2
NS = 16
C = 128
E_PAD = 327680
EPW = E_PAD // NS
NCHUNKS = EPW // C
NQ = 4
CQ = NCHUNKS // NQ
NPAIR_Q = CQ // 2
RPT = 632

_mesh = plsc.VectorSubcoreMesh(
    core_axis_name="c", subcore_axis_name="s", num_cores=NC, num_subcores=NS
)


@functools.partial(
    pl.kernel,
    out_type=jax.ShapeDtypeStruct((NC, N, DH), jnp.float32),
    mesh=_mesh,
    compiler_params=pltpu.CompilerParams(use_tc_tiling_on_sc=False),
    scratch_types=[
        pltpu.VMEM((CQ, C), jnp.int32),
        pltpu.VMEM((CQ, C), jnp.int32),
        pltpu.VMEM((CQ, C), jnp.float32),
        pltpu.VMEM((C, DH), jnp.float32),
        pltpu.VMEM((C, DH), jnp.float32),
        pltpu.VMEM_SHARED((N, DH), jnp.float32),
        pltpu.VMEM_SHARED((N, DH), jnp.float32),
        pltpu.SemaphoreType.DMA,
        pltpu.SemaphoreType.DMA,
        pltpu.SemaphoreType.DMA,
        pltpu.SemaphoreType.DMA,
    ],
)
def _spmm_sc(x_hbm, col_hbm, row_hbm, w_hbm, out_hbm, colv, rowv, wv,
             msg0, msg1, xs, acc, gsem0, gsem1, ssem0, ssem1):
    c = lax.axis_index("c")
    s = lax.axis_index("s")

    zeros16 = jnp.zeros((16,), jnp.float32)

    def _zrow(i, _):
        for j in range(DH // 16):
            msg0[i, pl.ds(j * 16, 16)] = zeros16
        return 0

    lax.fori_loop(0, C, _zrow, 0)

    row0 = s * RPT

    def _init_rows(nrows):
        pltpu.sync_copy(x_hbm.at[c].at[pl.ds(row0, nrows)],
                        xs.at[pl.ds(row0, nrows)])
        nfull = nrows // C
        for b in range(nfull):
            pltpu.sync_copy(msg0, acc.at[pl.ds(row0 + b * C, C)])
        rem = nrows - nfull * C
        if rem:
            pltpu.sync_copy(msg0.at[pl.ds(0, rem)],
                            acc.at[pl.ds(row0 + nfull * C, rem)])

    @pl.when(s < NS - 1)
    def _():
        _init_rows(RPT)

    @pl.when(s == NS - 1)
    def _():
        _init_rows(N - (NS - 1) * RPT)

    plsc.subcore_barrier()

    dnums = lax.GatherDimensionNumbers(
        offset_dims=(), collapsed_slice_dims=(0,), start_index_map=(0,))
    lane_idx = [jnp.full((16, 1), l, jnp.int32) for l in range(16)]

    def _scale(msg, k):
        def body(g, _):
            w16 = wv[k, pl.ds(g * 16, 16)]
            for l in range(16):
                i = g * 16 + l
                wb = lax.gather(w16, lane_idx[l], dnums, (1,),
                                mode=lax.GatherScatterMode.PROMISE_IN_BOUNDS)
                for j in range(DH // 16):
                    sl = pl.ds(j * 16, 16)
                    msg[i, sl] = msg[i, sl] * wb
            return 0

        lax.fori_loop(0, C // 16, body, 0)

    for q in range(NQ):
        pltpu.sync_copy(col_hbm.at[s].at[q], colv)
        pltpu.sync_copy(row_hbm.at[s].at[q], rowv)
        pltpu.sync_copy(w_hbm.at[s].at[q], wv)

        pltpu.async_copy(xs.at[colv.at[0]], msg0, gsem0)
        pltpu.async_copy(xs.at[colv.at[1]], msg1, gsem1)

        def _pair(t, _):
            k0 = 2 * t
            k1 = k0 + 1

            pltpu.make_async_copy(xs.at[colv.at[k0]], msg0, gsem0).wait()
            _scale(msg0, k0)
            scat0 = pltpu.async_copy(msg0, acc.at[rowv.at[k0]], ssem0,
                                     add=True)

            pltpu.make_async_copy(xs.at[colv.at[k1]], msg1, gsem1).wait()
            _scale(msg1, k1)
            scat1 = pltpu.async_copy(msg1, acc.at[rowv.at[k1]], ssem1,
                                     add=True)

            scat0.wait()
            scat1.wait()

            @pl.when(t + 1 < NPAIR_Q)
            def _prefetch():
                pltpu.async_copy(xs.at[colv.at[k0 + 2]], msg0, gsem0)
                pltpu.async_copy(xs.at[colv.at[k1 + 2]], msg1, gsem1)

            return 0

        lax.fori_loop(0, NPAIR_Q, _pair, 0)

    plsc.subcore_barrier()

    @pl.when(s < NS - 1)
    def _():
        pltpu.sync_copy(acc.at[pl.ds(row0, RPT)],
                        out_hbm.at[c].at[pl.ds(row0, RPT)])

    @pl.when(s == NS - 1)
    def _():
        last = N - (NS - 1) * RPT
        pltpu.sync_copy(acc.at[pl.ds(row0, last)],
                        out_hbm.at[c].at[pl.ds(row0, last)])


def kernel(x, edge_weight, edge_index):
    row = edge_index[0].astype(jnp.int32)
    col = edge_index[1].astype(jnp.int32)
    pad = E_PAD - E
    zi = jnp.zeros((pad,), jnp.int32)
    col = jnp.concatenate([col, zi]).reshape(NS, NQ, CQ, C)
    row = jnp.concatenate([row, zi]).reshape(NS, NQ, CQ, C)
    w = jnp.concatenate([edge_weight, jnp.zeros((pad,), jnp.float32)])
    w = w.reshape(NS, NQ, CQ, C)
    xh = x.reshape(N, NC, DH).transpose(1, 0, 2)
    partials = _spmm_sc(xh, col, row, w)
    return partials.transpose(1, 0, 2).reshape(N, D)


# speedup vs baseline: 1.6567x; 1.6567x over previous
import functools

import jax
import jax.numpy as jnp
from jax import lax
from jax.experimental import pallas as pl
from jax.experimental.pallas import tpu as pltpu
from jax.experimental.pallas import tpu_sc as plsc

N = 10000
D = 128
DH = D // 2
E = 320000
NC = 2
NS = 16
C = 128
E_PAD = 327680
EPW = E_PAD // NS
NCHUNKS = EPW // C
NQ = 4
CQ = NCHUNKS // NQ
NPAIR_Q = CQ // 2
RPT = 632

_mesh = plsc.VectorSubcoreMesh(
    core_axis_name="c", subcore_axis_name="s", num_cores=NC, num_subcores=NS
)


@functools.partial(
    pl.kernel,
    out_type=jax.ShapeDtypeStruct((NC, N, DH), jnp.float32),
    mesh=_mesh,
    compiler_params=pltpu.CompilerParams(use_tc_tiling_on_sc=False),
    scratch_types=[
        pltpu.VMEM((CQ, C), jnp.int32),
        pltpu.VMEM((CQ, C), jnp.int32),
        pltpu.VMEM((CQ, C), jnp.float32),
        pltpu.VMEM((C, DH), jnp.float32),
        pltpu.VMEM((C, DH), jnp.float32),
        pltpu.VMEM((C, DH), jnp.float32),
        pltpu.VMEM((C, DH), jnp.float32),
        pltpu.VMEM_SHARED((N, DH), jnp.float32),
        pltpu.VMEM_SHARED((N, DH), jnp.float32),
        pltpu.SemaphoreType.DMA,
        pltpu.SemaphoreType.DMA,
        pltpu.SemaphoreType.DMA,
        pltpu.SemaphoreType.DMA,
    ],
)
def _spmm_sc(x_hbm, col_hbm, row_hbm, w_hbm, out_hbm, colv, rowv, wv,
             msg0, msg1, msgo0, msgo1, xs, acc, gsem0, gsem1, ssem0, ssem1):
    c = lax.axis_index("c")
    s = lax.axis_index("s")

    zeros16 = jnp.zeros((16,), jnp.float32)

    def _zrow(i, _):
        for j in range(DH // 16):
            msg0[i, pl.ds(j * 16, 16)] = zeros16
        return 0

    lax.fori_loop(0, C, _zrow, 0)

    row0 = s * RPT

    def _init_rows(nrows):
        pltpu.sync_copy(x_hbm.at[c].at[pl.ds(row0, nrows)],
                        xs.at[pl.ds(row0, nrows)])
        nfull = nrows // C
        for b in range(nfull):
            pltpu.sync_copy(msg0, acc.at[pl.ds(row0 + b * C, C)])
        rem = nrows - nfull * C
        if rem:
            pltpu.sync_copy(msg0.at[pl.ds(0, rem)],
                            acc.at[pl.ds(row0 + nfull * C, rem)])

    @pl.when(s < NS - 1)
    def _():
        _init_rows(RPT)

    @pl.when(s == NS - 1)
    def _():
        _init_rows(N - (NS - 1) * RPT)

    plsc.subcore_barrier()

    dnums = lax.GatherDimensionNumbers(
        offset_dims=(), collapsed_slice_dims=(0,), start_index_map=(0,))
    lane_idx = [jnp.full((16, 1), l, jnp.int32) for l in range(16)]

    def _scale(msg, msgo, k):
        def body(g, _):
            w16 = wv[k, pl.ds(g * 16, 16)]
            for l in range(16):
                i = g * 16 + l
                wb = lax.gather(w16, lane_idx[l], dnums, (1,),
                                mode=lax.GatherScatterMode.PROMISE_IN_BOUNDS)
                for j in range(DH // 16):
                    sl = pl.ds(j * 16, 16)
                    msgo[i, sl] = msg[i, sl] * wb
            return 0

        lax.fori_loop(0, C // 16, body, 0)

    for q in range(NQ):
        pltpu.sync_copy(col_hbm.at[s].at[q], colv)
        pltpu.sync_copy(row_hbm.at[s].at[q], rowv)
        pltpu.sync_copy(w_hbm.at[s].at[q], wv)

        pltpu.async_copy(xs.at[colv.at[0]], msg0, gsem0)
        pltpu.async_copy(xs.at[colv.at[1]], msg1, gsem1)

        def _pair(t, _):
            k0 = 2 * t
            k1 = k0 + 1

            pltpu.make_async_copy(xs.at[colv.at[k0]], msg0, gsem0).wait()
            _scale(msg0, msgo0, k0)
            scat0 = pltpu.async_copy(msgo0, acc.at[rowv.at[k0]], ssem0,
                                     add=True)

            pltpu.make_async_copy(xs.at[colv.at[k1]], msg1, gsem1).wait()
            _scale(msg1, msgo1, k1)
            scat1 = pltpu.async_copy(msgo1, acc.at[rowv.at[k1]], ssem1,
                                     add=True)

            scat0.wait()
            scat1.wait()

            @pl.when(t + 1 < NPAIR_Q)
            def _prefetch():
                pltpu.async_copy(xs.at[colv.at[k0 + 2]], msg0, gsem0)
                pltpu.async_copy(xs.at[colv.at[k1 + 2]], msg1, gsem1)

            return 0

        lax.fori_loop(0, NPAIR_Q, _pair, 0)

    plsc.subcore_barrier()

    @pl.when(s < NS - 1)
    def _():
        pltpu.sync_copy(acc.at[pl.ds(row0, RPT)],
                        out_hbm.at[c].at[pl.ds(row0, RPT)])

    @pl.when(s == NS - 1)
    def _():
        last = N - (NS - 1) * RPT
        pltpu.sync_copy(acc.at[pl.ds(row0, last)],
                        out_hbm.at[c].at[pl.ds(row0, last)])


def kernel(x, edge_weight, edge_index):
    row = edge_index[0].astype(jnp.int32)
    col = edge_index[1].astype(jnp.int32)
    pad = E_PAD - E
    zi = jnp.zeros((pad,), jnp.int32)
    col = jnp.concatenate([col, zi]).reshape(NS, NQ, CQ, C)
    row = jnp.concatenate([row, zi]).reshape(NS, NQ, CQ, C)
    w = jnp.concatenate([edge_weight, jnp.zeros((pad,), jnp.float32)])
    w = w.reshape(NS, NQ, CQ, C)
    xh = x.reshape(N, NC, DH).transpose(1, 0, 2)
    partials = _spmm_sc(xh, col, row, w)
    return partials.transpose(1, 0, 2).reshape(N, D)


# speedup vs baseline: 2.3660x; 1.4282x over previous
import functools

import jax
import jax.numpy as jnp
from jax import lax
from jax.experimental import pallas as pl
from jax.experimental.pallas import tpu as pltpu
from jax.experimental.pallas import tpu_sc as plsc

N = 10000
D = 128
DH = D // 2
E = 320000
NC = 2
NS = 16
C = 64
E_PAD = 327680
EPW = E_PAD // NS
NCHUNKS = EPW // C
NQ = 4
CQ = NCHUNKS // NQ
NB = 4
ROUNDS = CQ // NB
RPT = 632

_mesh = plsc.VectorSubcoreMesh(
    core_axis_name="c", subcore_axis_name="s", num_cores=NC, num_subcores=NS
)


@functools.partial(
    pl.kernel,
    out_type=jax.ShapeDtypeStruct((NC, N, DH), jnp.float32),
    mesh=_mesh,
    compiler_params=pltpu.CompilerParams(use_tc_tiling_on_sc=False),
    scratch_types=[
        pltpu.VMEM((CQ, C), jnp.int32),
        pltpu.VMEM((CQ, C), jnp.int32),
        pltpu.VMEM((CQ, C), jnp.float32),
        [pltpu.VMEM((C, DH), jnp.float32) for _ in range(NB)],
        [pltpu.VMEM((C, DH), jnp.float32) for _ in range(NB)],
        pltpu.VMEM_SHARED((N, DH), jnp.float32),
        pltpu.VMEM_SHARED((N, DH), jnp.float32),
        [pltpu.SemaphoreType.DMA for _ in range(NB)],
        [pltpu.SemaphoreType.DMA for _ in range(NB)],
    ],
)
def _spmm_sc(x_hbm, col_hbm, row_hbm, w_hbm, out_hbm, colv, rowv, wv,
             msg, msgo, xs, acc, gsem, ssem):
    c = lax.axis_index("c")
    s = lax.axis_index("s")

    zeros16 = jnp.zeros((16,), jnp.float32)

    def _zrow(i, _):
        for j in range(DH // 16):
            msg[0][i, pl.ds(j * 16, 16)] = zeros16
        return 0

    lax.fori_loop(0, C, _zrow, 0)

    row0 = s * RPT

    def _init_rows(nrows):
        pltpu.sync_copy(x_hbm.at[c].at[pl.ds(row0, nrows)],
                        xs.at[pl.ds(row0, nrows)])
        nfull = nrows // C
        for b in range(nfull):
            pltpu.sync_copy(msg[0], acc.at[pl.ds(row0 + b * C, C)])
        rem = nrows - nfull * C
        if rem:
            pltpu.sync_copy(msg[0].at[pl.ds(0, rem)],
                            acc.at[pl.ds(row0 + nfull * C, rem)])

    @pl.when(s < NS - 1)
    def _():
        _init_rows(RPT)

    @pl.when(s == NS - 1)
    def _():
        _init_rows(N - (NS - 1) * RPT)

    plsc.subcore_barrier()

    dnums = lax.GatherDimensionNumbers(
        offset_dims=(), collapsed_slice_dims=(0,), start_index_map=(0,))
    lane_idx = [jnp.full((16, 1), l, jnp.int32) for l in range(16)]

    def _scale(mi, mo, k):
        def body(g, _):
            w16 = wv[k, pl.ds(g * 16, 16)]
            for l in range(16):
                i = g * 16 + l
                wb = lax.gather(w16, lane_idx[l], dnums, (1,),
                                mode=lax.GatherScatterMode.PROMISE_IN_BOUNDS)
                for j in range(DH // 16):
                    sl = pl.ds(j * 16, 16)
                    mo[i, sl] = mi[i, sl] * wb
            return 0

        lax.fori_loop(0, C // 16, body, 0)

    for q in range(NQ):
        pltpu.sync_copy(col_hbm.at[s].at[q], colv)
        pltpu.sync_copy(row_hbm.at[s].at[q], rowv)
        pltpu.sync_copy(w_hbm.at[s].at[q], wv)

        for b in range(NB):
            pltpu.async_copy(xs.at[colv.at[b]], msg[b], gsem[b])

        def _round(t, _):
            for b in range(NB):
                k = NB * t + b

                @pl.when(t > 0)
                def _():
                    pltpu.make_async_copy(msgo[b], acc.at[rowv.at[k - NB]],
                                          ssem[b]).wait()

                pltpu.make_async_copy(xs.at[colv.at[k]], msg[b],
                                      gsem[b]).wait()
                _scale(msg[b], msgo[b], k)
                pltpu.async_copy(msgo[b], acc.at[rowv.at[k]], ssem[b],
                                 add=True)

                @pl.when(t + 1 < ROUNDS)
                def _():
                    pltpu.async_copy(xs.at[colv.at[k + NB]], msg[b], gsem[b])

            return 0

        lax.fori_loop(0, ROUNDS, _round, 0)

        for b in range(NB):
            pltpu.make_async_copy(msgo[b],
                                  acc.at[rowv.at[NB * (ROUNDS - 1) + b]],
                                  ssem[b]).wait()

    plsc.subcore_barrier()

    @pl.when(s < NS - 1)
    def _():
        pltpu.sync_copy(acc.at[pl.ds(row0, RPT)],
                        out_hbm.at[c].at[pl.ds(row0, RPT)])

    @pl.when(s == NS - 1)
    def _():
        last = N - (NS - 1) * RPT
        pltpu.sync_copy(acc.at[pl.ds(row0, last)],
                        out_hbm.at[c].at[pl.ds(row0, last)])


def kernel(x, edge_weight, edge_index):
    row = edge_index[0].astype(jnp.int32)
    col = edge_index[1].astype(jnp.int32)
    pad = E_PAD - E
    zi = jnp.zeros((pad,), jnp.int32)
    col = jnp.concatenate([col, zi]).reshape(NS, NQ, CQ, C)
    row = jnp.concatenate([row, zi]).reshape(NS, NQ, CQ, C)
    w = jnp.concatenate([edge_weight, jnp.zeros((pad,), jnp.float32)])
    w = w.reshape(NS, NQ, CQ, C)
    xh = x.reshape(N, NC, DH).transpose(1, 0, 2)
    partials = _spmm_sc(xh, col, row, w)
    return partials.transpose(1, 0, 2).reshape(N, D)
